# baseline (speedup 1.0000x reference)
import jax
import jax.numpy as jnp
from jax.experimental import pallas as pl
from jax.experimental.pallas import tpu as pltpu

AFROMZERO = 0.02
AFROMC = 0.01
EXPMAX = 50.0
OVER_SAMPLE_FACTOR = 15.0

LBLK = 512


def _zmax_terms(a, b):
    c = 1.0
    zmax_zero = jnp.where(c > b, b / c, jnp.ones_like(b))
    zmax_aisc = b / (b + c)
    g = 0.5 * (b + c - jnp.sqrt((b - c) ** 2 + 4.0 * a * b)) / (c - a)
    g = jnp.where((g > 0.9999) & (b > 100.0), jnp.minimum(g, 1.0 - a / b), g)
    zmax = jnp.where(
        a < AFROMZERO, zmax_zero, jnp.where(jnp.abs(a - c) < AFROMC, zmax_aisc, g)
    )
    return 1.0 / zmax, jnp.log(zmax), jnp.log1p(-zmax)


def _body(mult_ref, pvec_ref, zt_ref, mt_ref, out_ref):
    e = pl.program_id(0)
    l = pl.program_id(1)
    m = mult_ref[e]

    @pl.when(l == 0)
    def _init():
        out_ref[...] = jnp.ones_like(out_ref)

    @pl.when(l * LBLK < m)
    def _compute():
        a_n = pvec_ref[0]
        b_n = pvec_ref[1]
        a_d = pvec_ref[2]
        b_d = pvec_ref[3]

        mt = mt_ref[0]
        zb = zt_ref[0]
        T = zb.shape[0]

        mt2 = mt * mt
        bn = b_n * mt2
        bd = b_d * mt2
        izn, lzn, l1zn = _zmax_terms(a_n, bn)
        izd, lzd, l1zd = _zmax_terms(a_d, bd)

        lane = jax.lax.broadcasted_iota(jnp.int32, (T, LBLK), 1)
        valid = (lane + l * LBLK) < m
        mask = valid & (zb != 0.0)
        zs = jnp.where(mask, zb, 0.5)

        rz = 1.0 / zs
        lz = jnp.log(zs)
        l1z = jnp.log1p(-zs)

        an_coef = jnp.where(a_n < AFROMZERO, 0.0, a_n * (l1z - l1zn))
        f_n = bn * (izn - rz) + (lzn - lz) + an_coef
        ad_coef = jnp.where(a_d < AFROMZERO, 0.0, a_d * (l1z - l1zd))
        f_d = bd * (izd - rz) + (lzd - lz) + ad_coef

        num = jnp.exp(jnp.clip(f_n, -EXPMAX, EXPMAX))
        den = jnp.exp(jnp.clip(f_d, -EXPMAX, EXPMAX))

        row = jax.lax.broadcasted_iota(jnp.int32, (T, LBLK), 0)
        wn = jnp.where(row == 0, num, OVER_SAMPLE_FACTOR - num)
        wd = jnp.where(row == 0, den, OVER_SAMPLE_FACTOR - den)
        w = jnp.where(mask, wn / wd, 1.0)

        k = T
        while k > 1:
            k //= 2
            w = w[:k] * w[k : 2 * k]
        width = LBLK
        while width > 128:
            half = width // 2
            w = w[:, :half] * w[:, half:width]
            width = half
        sh = 64
        while sh >= 1:
            w = w * pltpu.roll(w, sh, 1)
            sh //= 2
        out_ref[...] = out_ref[...] * w.reshape(1, 1, 128)


def _clamped_blk(l, mult_ref, e):
    nblk = jnp.maximum((mult_ref[e] + LBLK - 1) // LBLK, 1)
    return jnp.minimum(l, nblk - 1)


def _zt_map(e, l, mult_ref, pvec_ref):
    return (e, 0, _clamped_blk(l, mult_ref, e))


def _mt_map(e, l, mult_ref, pvec_ref):
    return (e, 0, _clamped_blk(l, mult_ref, e))


def _out_map(e, l, mult_ref, pvec_ref):
    return (e, 0, 0)


@jax.jit
def kernel(z, mT, observable, params_a, params_b, params_base):
    B, L, T = z.shape
    zt = jnp.transpose(z, (0, 2, 1))
    mT3 = mT.reshape(B, 1, L)
    mult = observable[:, 0]
    pvec = jnp.stack(
        [params_a, params_b, params_base[0], params_base[1]]
    ).astype(jnp.float32)
    nl = L // LBLK

    out = pl.pallas_call(
        _body,
        grid_spec=pltpu.PrefetchScalarGridSpec(
            num_scalar_prefetch=2,
            grid=(B, nl),
            in_specs=[
                pl.BlockSpec((1, T, LBLK), _zt_map),
                pl.BlockSpec((1, 1, LBLK), _mt_map),
            ],
            out_specs=pl.BlockSpec((1, 1, 128), _out_map),
        ),
        out_shape=jax.ShapeDtypeStruct((B, 1, 128), jnp.float32),
    )(mult, pvec, zt, mT3)
    return out[:, 0, 0]


# speedup vs baseline: 1.9374x; 1.9374x over previous
import jax
import jax.numpy as jnp
from jax.experimental import pallas as pl
from jax.experimental.pallas import tpu as pltpu

AFROMZERO = 0.02
AFROMC = 0.01
EXPMAX = 50.0
OVER_SAMPLE_FACTOR = 15.0

LBLK = 2048


def _zmax_terms(a, b):
    c = 1.0
    zmax_zero = jnp.where(c > b, b / c, jnp.ones_like(b))
    zmax_aisc = b / (b + c)
    g = 0.5 * (b + c - jnp.sqrt((b - c) ** 2 + 4.0 * a * b)) / (c - a)
    g = jnp.where((g > 0.9999) & (b > 100.0), jnp.minimum(g, 1.0 - a / b), g)
    zmax = jnp.where(
        a < AFROMZERO, zmax_zero, jnp.where(jnp.abs(a - c) < AFROMC, zmax_aisc, g)
    )
    return 1.0 / zmax, jnp.log(zmax), jnp.log1p(-zmax)


def _body(mult_ref, pvec_ref, zt_ref, mt_ref, out_ref):
    e = pl.program_id(0)
    l = pl.program_id(1)
    m = mult_ref[e]

    @pl.when(l == 0)
    def _init():
        out_ref[...] = jnp.ones_like(out_ref)

    @pl.when(l * LBLK < m)
    def _compute():
        a_n = pvec_ref[0]
        b_n = pvec_ref[1]
        a_d = pvec_ref[2]
        b_d = pvec_ref[3]

        mt = mt_ref[0]
        zb = zt_ref[0]
        T = zb.shape[0]

        mt2 = mt * mt
        bn = b_n * mt2
        bd = b_d * mt2
        izn, lzn, l1zn = _zmax_terms(a_n, bn)
        izd, lzd, l1zd = _zmax_terms(a_d, bd)

        lane = jax.lax.broadcasted_iota(jnp.int32, (T, LBLK), 1)
        valid = (lane + l * LBLK) < m
        mask = valid & (zb != 0.0)
        zs = jnp.where(mask, zb, 0.5)

        rz = 1.0 / zs
        lz = jnp.log(zs)
        l1z = jnp.log1p(-zs)

        an_coef = jnp.where(a_n < AFROMZERO, 0.0, a_n * (l1z - l1zn))
        f_n = bn * (izn - rz) + (lzn - lz) + an_coef
        ad_coef = jnp.where(a_d < AFROMZERO, 0.0, a_d * (l1z - l1zd))
        f_d = bd * (izd - rz) + (lzd - lz) + ad_coef

        num = jnp.exp(jnp.clip(f_n, -EXPMAX, EXPMAX))
        den = jnp.exp(jnp.clip(f_d, -EXPMAX, EXPMAX))

        row = jax.lax.broadcasted_iota(jnp.int32, (T, LBLK), 0)
        wn = jnp.where(row == 0, num, OVER_SAMPLE_FACTOR - num)
        wd = jnp.where(row == 0, den, OVER_SAMPLE_FACTOR - den)
        w = jnp.where(mask, wn / wd, 1.0)

        k = T
        while k > 1:
            k //= 2
            w = w[:k] * w[k : 2 * k]
        width = LBLK
        while width > 128:
            half = width // 2
            w = w[:, :half] * w[:, half:width]
            width = half
        sh = 64
        while sh >= 1:
            w = w * pltpu.roll(w, sh, 1)
            sh //= 2
        out_ref[...] = out_ref[...] * w.reshape(1, 1, 128)


def _clamped_blk(l, mult_ref, e):
    nblk = jnp.maximum((mult_ref[e] + LBLK - 1) // LBLK, 1)
    return jnp.minimum(l, nblk - 1)


def _zt_map(e, l, mult_ref, pvec_ref):
    return (e, 0, _clamped_blk(l, mult_ref, e))


def _mt_map(e, l, mult_ref, pvec_ref):
    return (e, 0, _clamped_blk(l, mult_ref, e))


def _out_map(e, l, mult_ref, pvec_ref):
    return (e, 0, 0)


@jax.jit
def kernel(z, mT, observable, params_a, params_b, params_base):
    B, L, T = z.shape
    zt = jnp.transpose(z, (0, 2, 1))
    mT3 = mT.reshape(B, 1, L)
    mult = observable[:, 0]
    pvec = jnp.stack(
        [params_a, params_b, params_base[0], params_base[1]]
    ).astype(jnp.float32)
    nl = L // LBLK

    out = pl.pallas_call(
        _body,
        grid_spec=pltpu.PrefetchScalarGridSpec(
            num_scalar_prefetch=2,
            grid=(B, nl),
            in_specs=[
                pl.BlockSpec((1, T, LBLK), _zt_map),
                pl.BlockSpec((1, 1, LBLK), _mt_map),
            ],
            out_specs=pl.BlockSpec((1, 1, 128), _out_map),
        ),
        out_shape=jax.ShapeDtypeStruct((B, 1, 128), jnp.float32),
    )(mult, pvec, zt, mT3)
    return out[:, 0, 0]


# speedup vs baseline: 2.9968x; 1.5468x over previous
import jax
import jax.numpy as jnp
from jax.experimental import pallas as pl
from jax.experimental.pallas import tpu as pltpu

AFROMZERO = 0.02
AFROMC = 0.01
EXPMAX = 50.0
OVER_SAMPLE_FACTOR = 15.0

WCH = 512
GRP = 1


def _zmax_calc(a, b):
    c = 1.0
    zmax_zero = jnp.where(c > b, b / c, jnp.ones_like(b))
    zmax_aisc = b / (b + c)
    g = 0.5 * (b + c - jnp.sqrt((b - c) ** 2 + 4.0 * a * b)) / (c - a)
    g = jnp.where((g > 0.9999) & (b > 100.0), jnp.minimum(g, 1.0 - a / b), g)
    return jnp.where(
        a < AFROMZERO, zmax_zero, jnp.where(jnp.abs(a - c) < AFROMC, zmax_aisc, g)
    )


def _body(
    mult_ref, pvec_ref, zt_ref, mt_ref, out_ref, acc_ref, tn_ref, td_ref, bn_ref, bd_ref
):
    e = pl.program_id(0)
    m = mult_ref[e]
    a_n = pvec_ref[0]
    b_n = pvec_ref[1]
    a_d = pvec_ref[2]
    b_d = pvec_ref[3]
    an_c = jnp.where(a_n < AFROMZERO, 0.0, a_n)
    ad_c = jnp.where(a_d < AFROMZERO, 0.0, a_d)

    T, L = zt_ref.shape[1], zt_ref.shape[2]
    nch = L // WCH

    mtp = mt_ref[0]
    mt2 = mtp * mtp
    bn = b_n * mt2
    bd = b_d * mt2
    zmn = _zmax_calc(a_n, bn)
    zmd = _zmax_calc(a_d, bd)
    tn_ref[...] = bn / zmn + jnp.log(zmn) - an_c * jnp.log(1.0 - zmn)
    td_ref[...] = bd / zmd + jnp.log(zmd) - ad_c * jnp.log(1.0 - zmd)
    bn_ref[...] = bn
    bd_ref[...] = bd

    acc_ref[...] = jnp.ones_like(acc_ref)

    for g in range(nch // GRP):

        @pl.when(g * GRP * WCH < m)
        def _group(g=g):
            pacc = None
            for i in range(g * GRP, (g + 1) * GRP):
                sl = slice(i * WCH, (i + 1) * WCH)
                zb = zt_ref[0, :, sl]

                lane = jax.lax.broadcasted_iota(jnp.int32, (1, WCH), 1)
                row0 = jax.lax.broadcasted_iota(jnp.int32, (T, WCH), 0) == 0
                mask = ((lane + i * WCH) < m) & (zb != 0.0)

                rz = 1.0 / zb
                lz = jnp.log(zb)
                l1z = jnp.log(1.0 - zb)

                f_n = tn_ref[i : i + 1, :] - bn_ref[i : i + 1, :] * rz - lz + an_c * l1z
                f_d = td_ref[i : i + 1, :] - bd_ref[i : i + 1, :] * rz - lz + ad_c * l1z

                num = jnp.exp(jnp.clip(f_n, -EXPMAX, EXPMAX))
                den = jnp.exp(jnp.clip(f_d, -EXPMAX, EXPMAX))

                wn = jnp.where(row0, num, OVER_SAMPLE_FACTOR - num)
                wd = jnp.where(row0, den, OVER_SAMPLE_FACTOR - den)
                w = jnp.where(mask, wn / wd, 1.0)

                width = WCH
                while width > 128:
                    half = width // 2
                    w = w[:, :half] * w[:, half:width]
                    width = half
                pacc = w if pacc is None else pacc * w
            acc_ref[...] = acc_ref[...] * pacc

    p = acc_ref[...]
    k = T
    while k > 1:
        k //= 2
        p = p[:k] * p[k : 2 * k]
    sh = 64
    while sh >= 1:
        p = p * pltpu.roll(p, sh, 1)
        sh //= 2
    out_ref[...] = p.reshape(1, 1, 128)


def _in_map(e, mult_ref, pvec_ref):
    return (e, 0, 0)


@jax.jit
def kernel(z, mT, observable, params_a, params_b, params_base):
    B, L, T = z.shape
    nch = L // WCH
    zt = jnp.transpose(z, (0, 2, 1))
    mTr = mT.reshape(B, nch, WCH)
    mult = observable[:, 0]
    pvec = jnp.stack(
        [params_a, params_b, params_base[0], params_base[1]]
    ).astype(jnp.float32)

    out = pl.pallas_call(
        _body,
        grid_spec=pltpu.PrefetchScalarGridSpec(
            num_scalar_prefetch=2,
            grid=(B,),
            in_specs=[
                pl.BlockSpec((1, T, L), _in_map),
                pl.BlockSpec((1, nch, WCH), _in_map),
            ],
            out_specs=pl.BlockSpec((1, 1, 128), _in_map),
            scratch_shapes=[
                pltpu.VMEM((T, 128), jnp.float32),
                pltpu.VMEM((nch, WCH), jnp.float32),
                pltpu.VMEM((nch, WCH), jnp.float32),
                pltpu.VMEM((nch, WCH), jnp.float32),
                pltpu.VMEM((nch, WCH), jnp.float32),
            ],
        ),
        out_shape=jax.ShapeDtypeStruct((B, 1, 128), jnp.float32),
    )(mult, pvec, zt, mTr)
    return out[:, 0, 0]
